# Initial kernel scaffold; baseline (speedup 1.0000x reference)
import jax
import jax.numpy as jnp
from jax.experimental import pallas as pl


def kernel(x, adj):
    raise NotImplementedError("write your pallas kernel here")


# speedup vs baseline: 1.0725x; 1.0725x over previous
import jax
import jax.numpy as jnp
from jax.experimental import pallas as pl
from jax.experimental.pallas import tpu as pltpu

_N = 4096
_BLOCK = 256

_KS0 = 0
_KS1 = 42
_KS2 = 0x1BD11BDA ^ _KS0 ^ _KS1
_KS = (_KS0, _KS1, _KS2)
_ROTS = ((13, 15, 26, 6), (17, 29, 16, 24))
_THRESH = 5872026


def _rotl(x, r):
    return (x << jnp.uint32(r)) | (x >> jnp.uint32(32 - r))


def _threefry_keep(flat_u32):
    x0 = jnp.full_like(flat_u32, jnp.uint32(_KS0))
    x1 = flat_u32 + jnp.uint32(_KS1)
    for i in range(5):
        for r in _ROTS[i % 2]:
            x0 = x0 + x1
            x1 = _rotl(x1, r) ^ x0
        x0 = x0 + jnp.uint32(_KS[(i + 1) % 3])
        x1 = x1 + jnp.uint32(_KS[(i + 2) % 3] + i + 1)
    bits = x0 ^ x1
    return (bits >> jnp.uint32(9)) < jnp.uint32(_THRESH)


def _dropedge_kernel(adj_ref, out_ref):
    g = pl.program_id(0)
    adj = adj_ref[...]
    b, n = adj.shape
    row = jax.lax.broadcasted_iota(jnp.uint32, (b, n), 0)
    col = jax.lax.broadcasted_iota(jnp.uint32, (b, n), 1)
    flat = (jnp.uint32(b) * g.astype(jnp.uint32) + row) * jnp.uint32(n) + col
    keep = _threefry_keep(flat)
    out_ref[...] = jnp.where(keep & (adj != 0.0), adj, 0.0)


def kernel(x, adj):
    t = pl.pallas_call(
        _dropedge_kernel,
        grid=(_N // _BLOCK,),
        in_specs=[pl.BlockSpec((_BLOCK, _N), lambda g: (g, 0))],
        out_specs=pl.BlockSpec((_BLOCK, _N), lambda g: (g, 0)),
        out_shape=jax.ShapeDtypeStruct((_N, _N), jnp.float32),
        compiler_params=pltpu.CompilerParams(
            dimension_semantics=("arbitrary",),
        ),
    )(adj)
    return (x, t)


# speedup vs baseline: 1.1972x; 1.1163x over previous
import jax
import jax.numpy as jnp
from jax.experimental import pallas as pl
from jax.experimental.pallas import tpu as pltpu
from jax.experimental.pallas import tpu_sc as plsc

_N = 4096
_ROWS_PER_WORD = 32
_NUM_WORD_ROWS = _N // _ROWS_PER_WORD
_TC_BLOCK = 256
_TC_GRID = _N // _TC_BLOCK

_NUM_WORKERS = 32
_WORDS_PER_WORKER = _NUM_WORD_ROWS * _N // _NUM_WORKERS

_KS = (0, 42, 0x1BD11BDA ^ 0 ^ 42)
_ROTS = ((13, 15, 26, 6), (17, 29, 16, 24))
_THRESH = 5872026


def _rotl(x, r):
    return (x << jnp.uint32(r)) | (x >> jnp.uint32(32 - r))


def _threefry_keep(flat_u32):
    x0 = jnp.full_like(flat_u32, jnp.uint32(_KS[0]))
    x1 = flat_u32 + jnp.uint32(_KS[1])
    for i in range(5):
        for r in _ROTS[i % 2]:
            x0 = x0 + x1
            x1 = _rotl(x1, r) ^ x0
        x0 = x0 + jnp.uint32(_KS[(i + 1) % 3])
        x1 = x1 + jnp.uint32(_KS[(i + 2) % 3] + i + 1)
    bits = x0 ^ x1
    return (bits >> jnp.uint32(9)) < jnp.uint32(_THRESH)


def _bitmask_kernel(adj_ref, mask_ref):
    riota = jax.lax.broadcasted_iota(jnp.int32, (_ROWS_PER_WORD, _N), 0)
    bitval = jnp.int32(1) << riota
    for w in range(_TC_BLOCK // _ROWS_PER_WORD):
        rows = adj_ref[pl.ds(_ROWS_PER_WORD * w, _ROWS_PER_WORD), :]
        bits = jnp.where(rows != 0.0, bitval, jnp.int32(0))
        mask_ref[w, :] = jnp.sum(bits, axis=0)


def _popcount16(v):
    c55 = jnp.full_like(v, 0x55555555)
    c33 = jnp.full_like(v, 0x33333333)
    c0f = jnp.full_like(v, 0x0F0F0F0F)
    v = v - (jax.lax.shift_right_logical(v, 1) & c55)
    v = (v & c33) + (jax.lax.shift_right_logical(v, 2) & c33)
    v = (v + jax.lax.shift_right_logical(v, 4)) & c0f
    return jax.lax.shift_right_logical(v * 0x01010101, 24)


def _lane_max(v, lane):
    for s in (8, 4, 2, 1):
        v = jnp.maximum(v, v.at[lane ^ s].get(mode="promise_in_bounds"))
    return v[0]


def _sc_drop_body(mask_hbm, kept_hbm, maskbuf, wordbuf):
    cid = jax.lax.axis_index("c")
    sid = jax.lax.axis_index("s")
    wid = sid * 2 + cid
    base_word = wid * _WORDS_PER_WORKER

    pltpu.sync_copy(mask_hbm.at[pl.ds(base_word, _WORDS_PER_WORKER)], maskbuf)

    lane = jax.lax.iota(jnp.int32, 16)
    w0 = base_word // _N

    @pl.loop(0, _WORDS_PER_WORKER // 16)
    def _vec(i):
        w = maskbuf[pl.ds(i * 16, 16)]
        lw = i * 16 + lane
        fbase = ((w0 + jax.lax.shift_right_logical(lw, 12)) * (32 * _N)
                 + (lw & (_N - 1)))
        rounds = _lane_max(_popcount16(w), lane)

        @pl.loop(0, rounds, init_carry=(w, jnp.zeros((16,), jnp.int32)))
        def _round(r, carry):
            wr, kept = carry
            lsb = wr & (0 - wr)
            live = lsb != 0
            bit = _popcount16(lsb - 1)
            flat = fbase + (bit << 12)
            keep = _threefry_keep(flat.astype(jnp.uint32))
            kept = kept | jnp.where(live & keep, lsb, 0)
            return (wr ^ lsb, kept)

        carry = _round
        wordbuf[pl.ds(i * 16, 16)] = carry[1]

    pltpu.sync_copy(wordbuf, kept_hbm.at[pl.ds(base_word, _WORDS_PER_WORKER)])


def _apply_kernel(adj_ref, kept_ref, out_ref):
    riota = jax.lax.broadcasted_iota(jnp.int32, (_ROWS_PER_WORD, _N), 0)
    one = jnp.int32(1)
    for w in range(_TC_BLOCK // _ROWS_PER_WORD):
        rows = adj_ref[pl.ds(_ROWS_PER_WORD * w, _ROWS_PER_WORD), :]
        word = kept_ref[w, :]
        bits = jax.lax.shift_right_logical(
            jnp.broadcast_to(word[None, :], (_ROWS_PER_WORD, _N)), riota) & one
        out_ref[pl.ds(_ROWS_PER_WORD * w, _ROWS_PER_WORD), :] = jnp.where(
            bits != 0, rows, 0.0)


def kernel(x, adj):
    mask = pl.pallas_call(
        _bitmask_kernel,
        grid=(_TC_GRID,),
        in_specs=[pl.BlockSpec((_TC_BLOCK, _N), lambda g: (g, 0))],
        out_specs=pl.BlockSpec((_TC_BLOCK // _ROWS_PER_WORD, _N),
                               lambda g: (g, 0)),
        out_shape=jax.ShapeDtypeStruct((_NUM_WORD_ROWS, _N), jnp.int32),
        compiler_params=pltpu.CompilerParams(
            dimension_semantics=("arbitrary",)),
    )(adj)

    kept_flat = pl.kernel(
        _sc_drop_body,
        out_type=jax.ShapeDtypeStruct((_NUM_WORD_ROWS * _N,), jnp.int32),
        mesh=plsc.VectorSubcoreMesh(core_axis_name="c", subcore_axis_name="s",
                                    num_cores=2, num_subcores=16),
        scratch_types=[
            pltpu.VMEM((_WORDS_PER_WORKER,), jnp.int32),
            pltpu.VMEM((_WORDS_PER_WORKER,), jnp.int32),
        ],
    )(jnp.reshape(mask, (_NUM_WORD_ROWS * _N,)))

    kept = jnp.reshape(kept_flat, (_NUM_WORD_ROWS, _N))

    t = pl.pallas_call(
        _apply_kernel,
        grid=(_TC_GRID,),
        in_specs=[
            pl.BlockSpec((_TC_BLOCK, _N), lambda g: (g, 0)),
            pl.BlockSpec((_TC_BLOCK // _ROWS_PER_WORD, _N), lambda g: (g, 0)),
        ],
        out_specs=pl.BlockSpec((_TC_BLOCK, _N), lambda g: (g, 0)),
        out_shape=jax.ShapeDtypeStruct((_N, _N), jnp.float32),
        compiler_params=pltpu.CompilerParams(
            dimension_semantics=("arbitrary",)),
    )(adj, kept)

    return (x, t)


# speedup vs baseline: 1.6735x; 1.3979x over previous
import jax
import jax.numpy as jnp
from jax.experimental import pallas as pl
from jax.experimental.pallas import tpu as pltpu
from jax.experimental.pallas import tpu_sc as plsc

_N = 4096
_ROWS_PER_WORD = 32
_NUM_WORD_ROWS = _N // _ROWS_PER_WORD
_TC_BLOCK = 256
_TC_GRID = _N // _TC_BLOCK

_NUM_WORKERS = 32
_WORDS_PER_WORKER = _NUM_WORD_ROWS * _N // _NUM_WORKERS

_KS = (0, 42, 0x1BD11BDA ^ 0 ^ 42)
_ROTS = ((13, 15, 26, 6), (17, 29, 16, 24))
_THRESH = 5872026


def _rotl(x, r):
    return (x << jnp.uint32(r)) | (x >> jnp.uint32(32 - r))


def _threefry_keep(flat_u32):
    x0 = jnp.full_like(flat_u32, jnp.uint32(_KS[0]))
    x1 = flat_u32 + jnp.uint32(_KS[1])
    for i in range(5):
        for r in _ROTS[i % 2]:
            x0 = x0 + x1
            x1 = _rotl(x1, r) ^ x0
        x0 = x0 + jnp.uint32(_KS[(i + 1) % 3])
        x1 = x1 + jnp.uint32(_KS[(i + 2) % 3] + i + 1)
    bits = x0 ^ x1
    return (bits >> jnp.uint32(9)) < jnp.uint32(_THRESH)


def _bitmask_kernel(adj_ref, mask_ref):
    riota = jax.lax.broadcasted_iota(jnp.int32, (_ROWS_PER_WORD, _N), 0)
    bitval = jnp.int32(1) << riota
    for w in range(_TC_BLOCK // _ROWS_PER_WORD):
        rows = adj_ref[pl.ds(_ROWS_PER_WORD * w, _ROWS_PER_WORD), :]
        bits = jnp.where(rows != 0.0, bitval, jnp.int32(0))
        mask_ref[w, :] = jnp.sum(bits, axis=0)


def _popcount16(v):
    c55 = jnp.full_like(v, 0x55555555)
    c33 = jnp.full_like(v, 0x33333333)
    c0f = jnp.full_like(v, 0x0F0F0F0F)
    v = v - (jax.lax.shift_right_logical(v, 1) & c55)
    v = (v & c33) + (jax.lax.shift_right_logical(v, 2) & c33)
    v = (v + jax.lax.shift_right_logical(v, 4)) & c0f
    return jax.lax.shift_right_logical(v * 0x01010101, 24)


def _lane_max(v, lane):
    for s in (8, 4, 2, 1):
        v = jnp.maximum(v, v.at[lane ^ s].get(mode="promise_in_bounds"))
    return v[0]


def _sc_drop_body(mask_hbm, kept_hbm, maskbuf, wordbuf):
    cid = jax.lax.axis_index("c")
    sid = jax.lax.axis_index("s")
    wid = sid * 2 + cid
    base_word = wid * _WORDS_PER_WORKER

    pltpu.sync_copy(mask_hbm.at[pl.ds(base_word, _WORDS_PER_WORKER)], maskbuf)

    lane = jax.lax.iota(jnp.int32, 16)
    w0 = base_word // _N
    _ILV = 4

    @pl.loop(0, _WORDS_PER_WORKER // (16 * _ILV))
    def _vec(i):
        ws, fbases = [], []
        pc = None
        for k in range(_ILV):
            w = maskbuf[pl.ds((i * _ILV + k) * 16, 16)]
            lw = (i * _ILV + k) * 16 + lane
            fbases.append((w0 + jax.lax.shift_right_logical(lw, 12))
                          * (32 * _N) + (lw & (_N - 1)))
            ws.append(w)
            p = _popcount16(w)
            pc = p if pc is None else jnp.maximum(pc, p)
        rounds = _lane_max(pc, lane)

        zero = jnp.zeros((16,), jnp.int32)
        init = tuple(ws) + (zero,) * _ILV

        @pl.loop(0, rounds, init_carry=init)
        def _round(r, carry):
            wr = list(carry[:_ILV])
            kept = list(carry[_ILV:])
            for k in range(_ILV):
                lsb = wr[k] & (0 - wr[k])
                live = lsb != 0
                bit = _popcount16(lsb - 1)
                flat = fbases[k] + (bit << 12)
                keep = _threefry_keep(flat.astype(jnp.uint32))
                kept[k] = kept[k] | jnp.where(live & keep, lsb, 0)
                wr[k] = wr[k] ^ lsb
            return tuple(wr) + tuple(kept)

        carry = _round
        for k in range(_ILV):
            wordbuf[pl.ds((i * _ILV + k) * 16, 16)] = carry[_ILV + k]

    pltpu.sync_copy(wordbuf, kept_hbm.at[pl.ds(base_word, _WORDS_PER_WORKER)])


def _apply_kernel(adj_ref, kept_ref, out_ref):
    riota = jax.lax.broadcasted_iota(jnp.int32, (_ROWS_PER_WORD, _N), 0)
    one = jnp.int32(1)
    for w in range(_TC_BLOCK // _ROWS_PER_WORD):
        rows = adj_ref[pl.ds(_ROWS_PER_WORD * w, _ROWS_PER_WORD), :]
        word = kept_ref[w, :]
        bits = jax.lax.shift_right_logical(
            jnp.broadcast_to(word[None, :], (_ROWS_PER_WORD, _N)), riota) & one
        out_ref[pl.ds(_ROWS_PER_WORD * w, _ROWS_PER_WORD), :] = jnp.where(
            bits != 0, rows, 0.0)


def kernel(x, adj):
    mask = pl.pallas_call(
        _bitmask_kernel,
        grid=(_TC_GRID,),
        in_specs=[pl.BlockSpec((_TC_BLOCK, _N), lambda g: (g, 0))],
        out_specs=pl.BlockSpec((_TC_BLOCK // _ROWS_PER_WORD, _N),
                               lambda g: (g, 0)),
        out_shape=jax.ShapeDtypeStruct((_NUM_WORD_ROWS, _N), jnp.int32),
        compiler_params=pltpu.CompilerParams(
            dimension_semantics=("arbitrary",)),
    )(adj)

    kept_flat = pl.kernel(
        _sc_drop_body,
        out_type=jax.ShapeDtypeStruct((_NUM_WORD_ROWS * _N,), jnp.int32),
        mesh=plsc.VectorSubcoreMesh(core_axis_name="c", subcore_axis_name="s",
                                    num_cores=2, num_subcores=16),
        scratch_types=[
            pltpu.VMEM((_WORDS_PER_WORKER,), jnp.int32),
            pltpu.VMEM((_WORDS_PER_WORKER,), jnp.int32),
        ],
    )(jnp.reshape(mask, (_NUM_WORD_ROWS * _N,)))

    kept = jnp.reshape(kept_flat, (_NUM_WORD_ROWS, _N))

    t = pl.pallas_call(
        _apply_kernel,
        grid=(_TC_GRID,),
        in_specs=[
            pl.BlockSpec((_TC_BLOCK, _N), lambda g: (g, 0)),
            pl.BlockSpec((_TC_BLOCK // _ROWS_PER_WORD, _N), lambda g: (g, 0)),
        ],
        out_specs=pl.BlockSpec((_TC_BLOCK, _N), lambda g: (g, 0)),
        out_shape=jax.ShapeDtypeStruct((_N, _N), jnp.float32),
        compiler_params=pltpu.CompilerParams(
            dimension_semantics=("arbitrary",)),
    )(adj, kept)

    return (x, t)
